# baseline (speedup 1.0000x reference)
import functools

import jax
import jax.numpy as jnp
from jax import lax
from jax.experimental import pallas as pl
from jax.experimental.pallas import tpu as pltpu
from jax.experimental.pallas import tpu_sc as plsc

B = 16384
D = 32
V = 1000
L = 20

_NC = 2
_NS = 16
_NW = _NC * _NS
_BPW = B // _NW
_CHUNK = 128
_NCHUNK = _BPW // _CHUNK


def _sc_gather_body(user_table, uids, item_table, iids, ue_out, ie_out,
                    idx_u, idx_i, rows_u, rows_i, sem_u, sem_i):
  wid = lax.axis_index("s") * _NC + lax.axis_index("c")
  base = wid * _BPW
  row0 = wid * _NCHUNK
  pltpu.sync_copy(uids.at[pl.ds(row0, _NCHUNK)], idx_u)
  pltpu.sync_copy(iids.at[pl.ds(row0, _NCHUNK)], idx_i)
  copies = []
  for j in range(_NCHUNK):
    copies.append(pltpu.async_copy(
        user_table.at[idx_u.at[j]], rows_u.at[pl.ds(j * _CHUNK, _CHUNK)],
        sem_u))
    copies.append(pltpu.async_copy(
        item_table.at[idx_i.at[j]], rows_i.at[pl.ds(j * _CHUNK, _CHUNK)],
        sem_i))
  for c in copies:
    c.wait()
  pltpu.sync_copy(rows_u, ue_out.at[pl.ds(base, _BPW)])
  pltpu.sync_copy(rows_i, ie_out.at[pl.ds(base, _BPW)])


@functools.partial(
    pl.kernel,
    out_type=(jax.ShapeDtypeStruct((B, D), jnp.float32),
              jax.ShapeDtypeStruct((B, D), jnp.float32)),
    mesh=plsc.VectorSubcoreMesh(core_axis_name="c", subcore_axis_name="s"),
    scratch_types=[
        pltpu.VMEM((_NCHUNK, _CHUNK), jnp.int32),
        pltpu.VMEM((_NCHUNK, _CHUNK), jnp.int32),
        pltpu.VMEM((_BPW, D), jnp.float32),
        pltpu.VMEM((_BPW, D), jnp.float32),
        pltpu.SemaphoreType.DMA,
        pltpu.SemaphoreType.DMA,
    ],
    compiler_params=pltpu.CompilerParams(use_tc_tiling_on_sc=False),
)
def _sc_gather(*args):
  _sc_gather_body(*args)


_TB = 1024


def _tc_body(lens_ref, ue_ref, ie_ref, wu_ref, wi_ref, logits_ref, mask_ref):
  dn = (((1,), (1,)), ((), ()))
  acc = lax.dot_general(ue_ref[...], wu_ref[...], dn,
                        preferred_element_type=jnp.float32)
  acc = acc + lax.dot_general(ie_ref[...], wi_ref[...], dn,
                              preferred_element_type=jnp.float32)
  logits_ref[...] = acc
  io = lax.broadcasted_iota(jnp.int32, (_TB, L), 1)
  mask_ref[...] = io >= lens_ref[...]


_tc_call = pl.pallas_call(
    _tc_body,
    grid=(B // _TB,),
    in_specs=[
        pl.BlockSpec((_TB, 1), lambda i: (i, 0)),
        pl.BlockSpec((_TB, D), lambda i: (i, 0)),
        pl.BlockSpec((_TB, D), lambda i: (i, 0)),
        pl.BlockSpec((V, D), lambda i: (0, 0)),
        pl.BlockSpec((V, D), lambda i: (0, 0)),
    ],
    out_specs=[
        pl.BlockSpec((_TB, V), lambda i: (i, 0)),
        pl.BlockSpec((_TB, L), lambda i: (i, 0)),
    ],
    out_shape=[
        jax.ShapeDtypeStruct((B, V), jnp.float32),
        jax.ShapeDtypeStruct((B, L), jnp.bool_),
    ],
)


def kernel(pos_attr_set, pos_attr_lens, neg_attr_set, neg_attr_lens,
           neg_attr_set_num, user_ids, item_ids, _, user_table, item_table,
           W_user, W_item):
  uids = user_ids.astype(jnp.int32).reshape(B // _CHUNK, _CHUNK)
  iids = item_ids.astype(jnp.int32).reshape(B // _CHUNK, _CHUNK)
  ue, ie = _sc_gather(user_table, uids, item_table, iids)
  logits, mask = _tc_call(pos_attr_lens.astype(jnp.int32).reshape(B, 1),
                          ue, ie, W_user, W_item)
  return (logits, mask)
